# Initial kernel scaffold; baseline (speedup 1.0000x reference)
import jax
import jax.numpy as jnp
from jax.experimental import pallas as pl


def kernel(x, edge_index, batch, gin1_W1, gin1_b1, gin1_W2, gin1_b2, gin2_W1, gin2_b1, gin2_W2, gin2_b2, gin3_W1, gin3_b1, gin3_W2, gin3_b2, gin4_W1, gin4_b1, gin4_W2, gin4_b2, gin5_W1, gin5_b1, gin5_W2, gin5_b2, lin_W, lin_b, ln_g, ln_b, p1_W, p1_b, p2_W, p2_b):
    raise NotImplementedError("write your pallas kernel here")


# speedup vs baseline: 2.6616x; 2.6616x over previous
import functools

import jax
import jax.numpy as jnp
from jax import lax
from jax.experimental import pallas as pl
from jax.experimental.pallas import tpu as pltpu
from jax.experimental.pallas import tpu_sc as plsc

N = 10000
E = 320000
G = 64
H = 256
OUT = 120
SLOPE = (1.0 / 8.0 + 1.0 / 3.0) / 2.0

TILES = 16
EB = 128
CH = 16
STEPS = 160
E_PAD = TILES * EB * STEPS
STEPS1 = 80
E_PAD1 = 2 * TILES * EB * STEPS1
AGG_ROWS = 10240
DUMMY = N + 8
ZSTRIPE = AGG_ROWS // TILES
OSTRIPE = 624
OLAST = N - 15 * OSTRIPE


def _make_sc_agg(mode):
    steps = STEPS if mode == "feat" else STEPS1
    mesh = plsc.VectorSubcoreMesh(core_axis_name="c", subcore_axis_name="s")

    @functools.partial(
        pl.kernel,
        mesh=mesh,
        out_type=jax.ShapeDtypeStruct((2, N, 128), jnp.float32),
        scratch_types=[
            pltpu.VMEM((CH, EB), jnp.int32),
            pltpu.VMEM((CH, EB), jnp.int32),
            pltpu.VMEM((EB, 128), jnp.float32),
            pltpu.VMEM_SHARED((AGG_ROWS, 128), jnp.float32),
            pltpu.SemaphoreType.DMA,
        ],
    )
    def k(xs_hbm, src_hbm, dst_hbm, out_hbm, src_v, dst_v, rows_v, agg_sh, sem):
        c = lax.axis_index("c")
        s = lax.axis_index("s")
        Dh = 128

        zero = jnp.zeros((16,), jnp.float32)

        def zb(i, carry):
            r = i // (Dh // 16)
            col = (i % (Dh // 16)) * 16
            rows_v[r, pl.ds(col, 16)] = zero
            return carry

        lax.fori_loop(0, EB * Dh // 16, zb, 0)

        base = s * ZSTRIPE
        for kk in range(ZSTRIPE // EB):
            pltpu.sync_copy(rows_v, agg_sh.at[pl.ds(base + kk * EB, EB)])
        plsc.subcore_barrier()

        def chunk(ci, carry):
            if mode == "feat":
                pltpu.sync_copy(src_hbm.at[c, s, pl.ds(ci * CH, CH)], src_v)
                pltpu.sync_copy(dst_hbm.at[s, pl.ds(ci * CH, CH)], dst_v)
            else:
                pltpu.sync_copy(src_hbm.at[c, s, pl.ds(ci * CH, CH)], src_v)
                pltpu.sync_copy(dst_hbm.at[c, s, pl.ds(ci * CH, CH)], dst_v)

            def body(j, carry2):
                pltpu.async_copy(xs_hbm.at[src_v.at[j]], rows_v, sem).wait()
                pltpu.sync_copy(rows_v, agg_sh.at[dst_v.at[j]], add=True)
                return carry2

            lax.fori_loop(0, CH, body, 0)
            return carry

        lax.fori_loop(0, steps // CH, chunk, 0)
        plsc.subcore_barrier()

        @pl.when(s < 15)
        def _():
            pltpu.sync_copy(agg_sh.at[pl.ds(s * OSTRIPE, OSTRIPE)],
                            out_hbm.at[c, pl.ds(s * OSTRIPE, OSTRIPE)])

        @pl.when(s == 15)
        def _():
            pltpu.sync_copy(agg_sh.at[pl.ds(15 * OSTRIPE, OLAST)],
                            out_hbm.at[c, pl.ds(15 * OSTRIPE, OLAST)])

    return k


_sc_agg_edge = _make_sc_agg("edge")
_sc_agg_feat = _make_sc_agg("feat")


def _mlp_call(xs, ag, W1, b1, W2, b2, first=False):
    D = W1.shape[0]
    BN = 1000
    grid = (N // BN,)

    def body(x_ref, a_ref, w1_ref, b1_ref, w2_ref, b2_ref, y_ref):
        if first:
            h = x_ref[...] + a_ref[0] + a_ref[1]
        else:
            xb = jnp.concatenate([x_ref[0], x_ref[1]], axis=1)
            ab = jnp.concatenate([a_ref[0], a_ref[1]], axis=1)
            h = xb + ab
        t = jnp.maximum(
            jnp.dot(h, w1_ref[...], preferred_element_type=jnp.float32)
            + b1_ref[...], 0.0)
        o = jnp.dot(t, w2_ref[...], preferred_element_type=jnp.float32) \
            + b2_ref[...]
        o = jnp.where(o >= 0, o, o * SLOPE)
        y_ref[0] = o[:, :H // 2]
        y_ref[1] = o[:, H // 2:]

    x_spec = (pl.BlockSpec((BN, 128), lambda i: (i, 0)) if first
              else pl.BlockSpec((2, BN, 128), lambda i: (0, i, 0)))
    return pl.pallas_call(
        body,
        grid=grid,
        in_specs=[
            x_spec,
            pl.BlockSpec((2, BN, 128), lambda i: (0, i, 0)),
            pl.BlockSpec((D, H), lambda i: (0, 0)),
            pl.BlockSpec((1, H), lambda i: (0, 0)),
            pl.BlockSpec((H, H), lambda i: (0, 0)),
            pl.BlockSpec((1, H), lambda i: (0, 0)),
        ],
        out_specs=pl.BlockSpec((2, BN, H // 2), lambda i: (0, i, 0)),
        out_shape=jax.ShapeDtypeStruct((2, N, H // 2), jnp.float32),
    )(xs, ag, W1, b1.reshape(1, H), W2, b2.reshape(1, H))


def _final_call(xs, batch2d, lin_W, lin_b, ln_g, ln_b, p1_W, p1_b, p2_W, p2_b):
    BN = 1000
    GRID = N // BN

    def body(x_ref, b_ref, linW_ref, linb_ref, lng_ref, lnb_ref,
             p1W_ref, p1b_ref, p2W_ref, p2b_ref, z_ref, pooled, cnt):
        i = pl.program_id(0)
        xb = jnp.concatenate([x_ref[0], x_ref[1]], axis=1)
        oh = (b_ref[...] == lax.broadcasted_iota(jnp.int32, (BN, G), 1)
              ).astype(jnp.float32)
        pooled_blk = lax.dot_general(
            oh, xb, (((0,), (0,)), ((), ())),
            preferred_element_type=jnp.float32)
        cnt_blk = lax.dot_general(
            oh, jnp.ones((BN, 8), jnp.float32), (((0,), (0,)), ((), ())),
            preferred_element_type=jnp.float32)

        @pl.when(i == 0)
        def _():
            pooled[...] = pooled_blk
            cnt[...] = cnt_blk

        @pl.when(i > 0)
        def _():
            pooled[...] += pooled_blk
            cnt[...] += cnt_blk

        @pl.when(i == GRID - 1)
        def _():
            p = jnp.dot(pooled[...], linW_ref[...],
                        preferred_element_type=jnp.float32) \
                + cnt[...][:, :1] * linb_ref[...]
            mu = jnp.mean(p, axis=1, keepdims=True)
            var = jnp.mean((p - mu) ** 2, axis=1, keepdims=True)
            hh = (p - mu) / jnp.sqrt(var + 1e-5) * lng_ref[...] + lnb_ref[...]
            a = jnp.dot(hh, p1W_ref[...],
                        preferred_element_type=jnp.float32) + p1b_ref[...]
            a = jnp.where(a >= 0, a, a * SLOPE)
            z_ref[...] = jnp.dot(a, p2W_ref[...],
                                 preferred_element_type=jnp.float32) \
                + p2b_ref[...]

    return pl.pallas_call(
        body,
        grid=(GRID,),
        in_specs=[
            pl.BlockSpec((2, BN, H // 2), lambda i: (0, i, 0)),
            pl.BlockSpec((BN, 1), lambda i: (i, 0)),
            pl.BlockSpec((H, OUT), lambda i: (0, 0)),
            pl.BlockSpec((1, OUT), lambda i: (0, 0)),
            pl.BlockSpec((1, OUT), lambda i: (0, 0)),
            pl.BlockSpec((1, OUT), lambda i: (0, 0)),
            pl.BlockSpec((OUT, 256), lambda i: (0, 0)),
            pl.BlockSpec((1, 256), lambda i: (0, 0)),
            pl.BlockSpec((256, OUT), lambda i: (0, 0)),
            pl.BlockSpec((1, OUT), lambda i: (0, 0)),
        ],
        out_specs=pl.BlockSpec((G, OUT), lambda i: (0, 0)),
        out_shape=jax.ShapeDtypeStruct((G, OUT), jnp.float32),
        scratch_shapes=[
            pltpu.VMEM((G, 256), jnp.float32),
            pltpu.VMEM((G, 8), jnp.float32),
        ],
    )(xs, batch2d, lin_W, lin_b.reshape(1, OUT), ln_g.reshape(1, OUT),
      ln_b.reshape(1, OUT), p1_W, p1_b.reshape(1, 256), p2_W,
      p2_b.reshape(1, OUT))


def kernel(x, edge_index, batch,
           gin1_W1, gin1_b1, gin1_W2, gin1_b2,
           gin2_W1, gin2_b1, gin2_W2, gin2_b2,
           gin3_W1, gin3_b1, gin3_W2, gin3_b2,
           gin4_W1, gin4_b1, gin4_W2, gin4_b2,
           gin5_W1, gin5_b1, gin5_W2, gin5_b2,
           lin_W, lin_b, ln_g, ln_b,
           p1_W, p1_b, p2_W, p2_b):
    src = edge_index[0]
    dst = edge_index[1]
    pad1 = E_PAD1 - E
    src1 = jnp.concatenate([src, jnp.zeros((pad1,), jnp.int32)])
    dst1 = jnp.concatenate([dst, jnp.full((pad1,), DUMMY, jnp.int32)])
    src1 = src1.reshape(2, TILES, STEPS1, EB)
    dst1 = dst1.reshape(2, TILES, STEPS1, EB)
    pad = E_PAD - E
    src_p = jnp.concatenate([src, jnp.zeros((pad,), jnp.int32)])
    dst_p = jnp.concatenate([dst, jnp.full((pad,), DUMMY, jnp.int32)])
    src2 = jnp.stack([src_p, src_p + N]).reshape(2, TILES, STEPS, EB)
    dst2 = dst_p.reshape(TILES, STEPS, EB)

    ag = _sc_agg_edge(x, src1, dst1)
    xs = _mlp_call(x, ag, gin1_W1, gin1_b1, gin1_W2, gin1_b2, first=True)

    for (W1, b1, W2, b2) in ((gin2_W1, gin2_b1, gin2_W2, gin2_b2),
                             (gin3_W1, gin3_b1, gin3_W2, gin3_b2),
                             (gin4_W1, gin4_b1, gin4_W2, gin4_b2),
                             (gin5_W1, gin5_b1, gin5_W2, gin5_b2)):
        ag = _sc_agg_feat(xs.reshape(2 * N, 128), src2, dst2)
        xs = _mlp_call(xs, ag, W1, b1, W2, b2)

    return _final_call(xs, batch.reshape(N, 1), lin_W, lin_b, ln_g, ln_b,
                       p1_W, p1_b, p2_W, p2_b)


# speedup vs baseline: 3.1433x; 1.1810x over previous
import functools

import jax
import jax.numpy as jnp
from jax import lax
from jax.experimental import pallas as pl
from jax.experimental.pallas import tpu as pltpu
from jax.experimental.pallas import tpu_sc as plsc

N = 10000
E = 320000
G = 64
H = 256
OUT = 120
SLOPE = (1.0 / 8.0 + 1.0 / 3.0) / 2.0

TILES = 16
EB = 128
CH = 16
STEPS = 160
E_PAD = TILES * EB * STEPS
STEPS1 = 80
E_PAD1 = 2 * TILES * EB * STEPS1
AGG_ROWS = 10240
DUMMY = N + 8
ZSTRIPE = AGG_ROWS // TILES
OSTRIPE = 624
OLAST = N - 15 * OSTRIPE


def _make_sc_agg(mode):
    steps = STEPS if mode == "feat" else STEPS1
    mesh = plsc.VectorSubcoreMesh(core_axis_name="c", subcore_axis_name="s")

    @functools.partial(
        pl.kernel,
        mesh=mesh,
        out_type=jax.ShapeDtypeStruct((2, N, 128), jnp.float32),
        scratch_types=[
            pltpu.VMEM((CH, EB), jnp.int32),
            pltpu.VMEM((CH, EB), jnp.int32),
            pltpu.VMEM((EB, 128), jnp.float32),
            pltpu.VMEM((EB, 128), jnp.float32),
            pltpu.VMEM_SHARED((AGG_ROWS, 128), jnp.float32),
            pltpu.SemaphoreType.DMA,
            pltpu.SemaphoreType.DMA,
        ],
    )
    def k(xs_hbm, src_hbm, dst_hbm, out_hbm, src_v, dst_v, rows_a, rows_b,
          agg_sh, sem_a, sem_b):
        rows_v = rows_a
        c = lax.axis_index("c")
        s = lax.axis_index("s")
        Dh = 128

        zero = jnp.zeros((16,), jnp.float32)

        def zb(i, carry):
            r = i // (Dh // 16)
            col = (i % (Dh // 16)) * 16
            rows_v[r, pl.ds(col, 16)] = zero
            return carry

        lax.fori_loop(0, EB * Dh // 16, zb, 0)

        base = s * ZSTRIPE
        for kk in range(ZSTRIPE // EB):
            pltpu.sync_copy(rows_v, agg_sh.at[pl.ds(base + kk * EB, EB)])
        plsc.subcore_barrier()

        def chunk(ci, carry):
            if mode == "feat":
                pltpu.sync_copy(src_hbm.at[c, s, pl.ds(ci * CH, CH)], src_v)
                pltpu.sync_copy(dst_hbm.at[s, pl.ds(ci * CH, CH)], dst_v)
            else:
                pltpu.sync_copy(src_hbm.at[c, s, pl.ds(ci * CH, CH)], src_v)
                pltpu.sync_copy(dst_hbm.at[c, s, pl.ds(ci * CH, CH)], dst_v)

            pltpu.async_copy(xs_hbm.at[src_v.at[0]], rows_a, sem_a)

            def body(j, carry2):
                nxt = j + 1

                @pl.when(jnp.logical_and(nxt < CH, nxt % 2 == 1))
                def _():
                    pltpu.async_copy(xs_hbm.at[src_v.at[nxt]], rows_b, sem_b)

                @pl.when(jnp.logical_and(nxt < CH, nxt % 2 == 0))
                def _():
                    pltpu.async_copy(xs_hbm.at[src_v.at[nxt]], rows_a, sem_a)

                @pl.when(j % 2 == 0)
                def _():
                    pltpu.make_async_copy(xs_hbm.at[src_v.at[j]], rows_a,
                                          sem_a).wait()
                    pltpu.sync_copy(rows_a, agg_sh.at[dst_v.at[j]], add=True)

                @pl.when(j % 2 == 1)
                def _():
                    pltpu.make_async_copy(xs_hbm.at[src_v.at[j]], rows_b,
                                          sem_b).wait()
                    pltpu.sync_copy(rows_b, agg_sh.at[dst_v.at[j]], add=True)

                return carry2

            lax.fori_loop(0, CH, body, 0)
            return carry

        lax.fori_loop(0, steps // CH, chunk, 0)
        plsc.subcore_barrier()

        @pl.when(s < 15)
        def _():
            pltpu.sync_copy(agg_sh.at[pl.ds(s * OSTRIPE, OSTRIPE)],
                            out_hbm.at[c, pl.ds(s * OSTRIPE, OSTRIPE)])

        @pl.when(s == 15)
        def _():
            pltpu.sync_copy(agg_sh.at[pl.ds(15 * OSTRIPE, OLAST)],
                            out_hbm.at[c, pl.ds(15 * OSTRIPE, OLAST)])

    return k


_sc_agg_edge = _make_sc_agg("edge")
_sc_agg_feat = _make_sc_agg("feat")


def _mlp_call(xs, ag, W1, b1, W2, b2, first=False):
    D = W1.shape[0]
    BN = 1000
    grid = (N // BN,)

    def body(x_ref, a_ref, w1_ref, b1_ref, w2_ref, b2_ref, y_ref):
        if first:
            h = x_ref[...] + a_ref[0] + a_ref[1]
        else:
            xb = jnp.concatenate([x_ref[0], x_ref[1]], axis=1)
            ab = jnp.concatenate([a_ref[0], a_ref[1]], axis=1)
            h = xb + ab
        t = jnp.maximum(
            jnp.dot(h, w1_ref[...], preferred_element_type=jnp.float32)
            + b1_ref[...], 0.0)
        o = jnp.dot(t, w2_ref[...], preferred_element_type=jnp.float32) \
            + b2_ref[...]
        o = jnp.where(o >= 0, o, o * SLOPE)
        y_ref[0] = o[:, :H // 2]
        y_ref[1] = o[:, H // 2:]

    x_spec = (pl.BlockSpec((BN, 128), lambda i: (i, 0)) if first
              else pl.BlockSpec((2, BN, 128), lambda i: (0, i, 0)))
    return pl.pallas_call(
        body,
        grid=grid,
        in_specs=[
            x_spec,
            pl.BlockSpec((2, BN, 128), lambda i: (0, i, 0)),
            pl.BlockSpec((D, H), lambda i: (0, 0)),
            pl.BlockSpec((1, H), lambda i: (0, 0)),
            pl.BlockSpec((H, H), lambda i: (0, 0)),
            pl.BlockSpec((1, H), lambda i: (0, 0)),
        ],
        out_specs=pl.BlockSpec((2, BN, H // 2), lambda i: (0, i, 0)),
        out_shape=jax.ShapeDtypeStruct((2, N, H // 2), jnp.float32),
    )(xs, ag, W1, b1.reshape(1, H), W2, b2.reshape(1, H))


def _final_call(xs, batch2d, lin_W, lin_b, ln_g, ln_b, p1_W, p1_b, p2_W, p2_b):
    BN = 1000
    GRID = N // BN

    def body(x_ref, b_ref, linW_ref, linb_ref, lng_ref, lnb_ref,
             p1W_ref, p1b_ref, p2W_ref, p2b_ref, z_ref, pooled, cnt):
        i = pl.program_id(0)
        xb = jnp.concatenate([x_ref[0], x_ref[1]], axis=1)
        oh = (b_ref[...] == lax.broadcasted_iota(jnp.int32, (BN, G), 1)
              ).astype(jnp.float32)
        pooled_blk = lax.dot_general(
            oh, xb, (((0,), (0,)), ((), ())),
            preferred_element_type=jnp.float32)
        cnt_blk = lax.dot_general(
            oh, jnp.ones((BN, 8), jnp.float32), (((0,), (0,)), ((), ())),
            preferred_element_type=jnp.float32)

        @pl.when(i == 0)
        def _():
            pooled[...] = pooled_blk
            cnt[...] = cnt_blk

        @pl.when(i > 0)
        def _():
            pooled[...] += pooled_blk
            cnt[...] += cnt_blk

        @pl.when(i == GRID - 1)
        def _():
            p = jnp.dot(pooled[...], linW_ref[...],
                        preferred_element_type=jnp.float32) \
                + cnt[...][:, :1] * linb_ref[...]
            mu = jnp.mean(p, axis=1, keepdims=True)
            var = jnp.mean((p - mu) ** 2, axis=1, keepdims=True)
            hh = (p - mu) / jnp.sqrt(var + 1e-5) * lng_ref[...] + lnb_ref[...]
            a = jnp.dot(hh, p1W_ref[...],
                        preferred_element_type=jnp.float32) + p1b_ref[...]
            a = jnp.where(a >= 0, a, a * SLOPE)
            z_ref[...] = jnp.dot(a, p2W_ref[...],
                                 preferred_element_type=jnp.float32) \
                + p2b_ref[...]

    return pl.pallas_call(
        body,
        grid=(GRID,),
        in_specs=[
            pl.BlockSpec((2, BN, H // 2), lambda i: (0, i, 0)),
            pl.BlockSpec((BN, 1), lambda i: (i, 0)),
            pl.BlockSpec((H, OUT), lambda i: (0, 0)),
            pl.BlockSpec((1, OUT), lambda i: (0, 0)),
            pl.BlockSpec((1, OUT), lambda i: (0, 0)),
            pl.BlockSpec((1, OUT), lambda i: (0, 0)),
            pl.BlockSpec((OUT, 256), lambda i: (0, 0)),
            pl.BlockSpec((1, 256), lambda i: (0, 0)),
            pl.BlockSpec((256, OUT), lambda i: (0, 0)),
            pl.BlockSpec((1, OUT), lambda i: (0, 0)),
        ],
        out_specs=pl.BlockSpec((G, OUT), lambda i: (0, 0)),
        out_shape=jax.ShapeDtypeStruct((G, OUT), jnp.float32),
        scratch_shapes=[
            pltpu.VMEM((G, 256), jnp.float32),
            pltpu.VMEM((G, 8), jnp.float32),
        ],
    )(xs, batch2d, lin_W, lin_b.reshape(1, OUT), ln_g.reshape(1, OUT),
      ln_b.reshape(1, OUT), p1_W, p1_b.reshape(1, 256), p2_W,
      p2_b.reshape(1, OUT))


def kernel(x, edge_index, batch,
           gin1_W1, gin1_b1, gin1_W2, gin1_b2,
           gin2_W1, gin2_b1, gin2_W2, gin2_b2,
           gin3_W1, gin3_b1, gin3_W2, gin3_b2,
           gin4_W1, gin4_b1, gin4_W2, gin4_b2,
           gin5_W1, gin5_b1, gin5_W2, gin5_b2,
           lin_W, lin_b, ln_g, ln_b,
           p1_W, p1_b, p2_W, p2_b):
    src = edge_index[0]
    dst = edge_index[1]
    pad1 = E_PAD1 - E
    src1 = jnp.concatenate([src, jnp.zeros((pad1,), jnp.int32)])
    dst1 = jnp.concatenate([dst, jnp.full((pad1,), DUMMY, jnp.int32)])
    src1 = src1.reshape(2, TILES, STEPS1, EB)
    dst1 = dst1.reshape(2, TILES, STEPS1, EB)
    pad = E_PAD - E
    src_p = jnp.concatenate([src, jnp.zeros((pad,), jnp.int32)])
    dst_p = jnp.concatenate([dst, jnp.full((pad,), DUMMY, jnp.int32)])
    src2 = jnp.stack([src_p, src_p + N]).reshape(2, TILES, STEPS, EB)
    dst2 = dst_p.reshape(TILES, STEPS, EB)

    ag = _sc_agg_edge(x, src1, dst1)
    xs = _mlp_call(x, ag, gin1_W1, gin1_b1, gin1_W2, gin1_b2, first=True)

    for (W1, b1, W2, b2) in ((gin2_W1, gin2_b1, gin2_W2, gin2_b2),
                             (gin3_W1, gin3_b1, gin3_W2, gin3_b2),
                             (gin4_W1, gin4_b1, gin4_W2, gin4_b2),
                             (gin5_W1, gin5_b1, gin5_W2, gin5_b2)):
        ag = _sc_agg_feat(xs.reshape(2 * N, 128), src2, dst2)
        xs = _mlp_call(xs, ag, W1, b1, W2, b2)

    return _final_call(xs, batch.reshape(N, 1), lin_W, lin_b, ln_g, ln_b,
                       p1_W, p1_b, p2_W, p2_b)


# speedup vs baseline: 3.1722x; 1.0092x over previous
import functools

import jax
import jax.numpy as jnp
from jax import lax
from jax.experimental import pallas as pl
from jax.experimental.pallas import tpu as pltpu
from jax.experimental.pallas import tpu_sc as plsc

N = 10000
E = 320000
G = 64
H = 256
OUT = 120
SLOPE = (1.0 / 8.0 + 1.0 / 3.0) / 2.0

TILES = 16
EB = 64
CH = 16
NBUF = 4
STEPS = 320
E_PAD = TILES * EB * STEPS
STEPS1 = 160
E_PAD1 = 2 * TILES * EB * STEPS1
AGG_ROWS = 10240
DUMMY = N + 8
ZSTRIPE = AGG_ROWS // TILES
OSTRIPE = 624
OLAST = N - 15 * OSTRIPE


def _make_sc_agg(mode):
    steps = STEPS if mode == "feat" else STEPS1
    mesh = plsc.VectorSubcoreMesh(core_axis_name="c", subcore_axis_name="s")

    @functools.partial(
        pl.kernel,
        mesh=mesh,
        out_type=jax.ShapeDtypeStruct((2, N, 128), jnp.float32),
        scratch_types=[
            pltpu.VMEM((CH, EB), jnp.int32),
            pltpu.VMEM((CH, EB), jnp.int32),
            pltpu.VMEM((EB, 128), jnp.float32),
            pltpu.VMEM((EB, 128), jnp.float32),
            pltpu.VMEM((EB, 128), jnp.float32),
            pltpu.VMEM((EB, 128), jnp.float32),
            pltpu.VMEM_SHARED((AGG_ROWS, 128), jnp.float32),
            pltpu.SemaphoreType.DMA,
            pltpu.SemaphoreType.DMA,
            pltpu.SemaphoreType.DMA,
            pltpu.SemaphoreType.DMA,
        ],
    )
    def k(xs_hbm, src_hbm, dst_hbm, out_hbm, src_v, dst_v, rows_0, rows_1,
          rows_2, rows_3, agg_sh, sem_0, sem_1, sem_2, sem_3):
        rows = (rows_0, rows_1, rows_2, rows_3)
        sems = (sem_0, sem_1, sem_2, sem_3)
        rows_v = rows_0
        c = lax.axis_index("c")
        s = lax.axis_index("s")
        Dh = 128

        zero = jnp.zeros((16,), jnp.float32)

        def zb(i, carry):
            r = i // (Dh // 16)
            col = (i % (Dh // 16)) * 16
            rows_v[r, pl.ds(col, 16)] = zero
            return carry

        lax.fori_loop(0, EB * Dh // 16, zb, 0)

        base = s * ZSTRIPE
        for kk in range(ZSTRIPE // EB):
            pltpu.sync_copy(rows_v, agg_sh.at[pl.ds(base + kk * EB, EB)])
        plsc.subcore_barrier()

        def chunk(ci, carry):
            if mode == "feat":
                pltpu.sync_copy(src_hbm.at[c, s, pl.ds(ci * CH, CH)], src_v)
                pltpu.sync_copy(dst_hbm.at[s, pl.ds(ci * CH, CH)], dst_v)
            else:
                pltpu.sync_copy(src_hbm.at[c, s, pl.ds(ci * CH, CH)], src_v)
                pltpu.sync_copy(dst_hbm.at[c, s, pl.ds(ci * CH, CH)], dst_v)

            for p in range(NBUF - 1):
                pltpu.async_copy(xs_hbm.at[src_v.at[p]], rows[p], sems[p])

            def body(j, carry2):
                for slot in range(NBUF):
                    @pl.when(j % NBUF == slot)
                    def _(slot=slot):
                        pltpu.make_async_copy(xs_hbm.at[src_v.at[j]],
                                              rows[slot], sems[slot]).wait()
                        nxt = j + NBUF - 1
                        ns = (slot + NBUF - 1) % NBUF

                        @pl.when(nxt < CH)
                        def _():
                            pltpu.async_copy(xs_hbm.at[src_v.at[nxt]],
                                             rows[ns], sems[ns])

                        pltpu.sync_copy(rows[slot], agg_sh.at[dst_v.at[j]],
                                        add=True)

                return carry2

            lax.fori_loop(0, CH, body, 0)
            return carry

        lax.fori_loop(0, steps // CH, chunk, 0)
        plsc.subcore_barrier()

        @pl.when(s < 15)
        def _():
            pltpu.sync_copy(agg_sh.at[pl.ds(s * OSTRIPE, OSTRIPE)],
                            out_hbm.at[c, pl.ds(s * OSTRIPE, OSTRIPE)])

        @pl.when(s == 15)
        def _():
            pltpu.sync_copy(agg_sh.at[pl.ds(15 * OSTRIPE, OLAST)],
                            out_hbm.at[c, pl.ds(15 * OSTRIPE, OLAST)])

    return k


_sc_agg_edge = _make_sc_agg("edge")
_sc_agg_feat = _make_sc_agg("feat")


def _mlp_call(xs, ag, W1, b1, W2, b2, first=False):
    D = W1.shape[0]
    BN = 1000
    grid = (N // BN,)

    def body(x_ref, a_ref, w1_ref, b1_ref, w2_ref, b2_ref, y_ref):
        if first:
            h = x_ref[...] + a_ref[0] + a_ref[1]
        else:
            xb = jnp.concatenate([x_ref[0], x_ref[1]], axis=1)
            ab = jnp.concatenate([a_ref[0], a_ref[1]], axis=1)
            h = xb + ab
        t = jnp.maximum(
            jnp.dot(h, w1_ref[...], preferred_element_type=jnp.float32)
            + b1_ref[...], 0.0)
        o = jnp.dot(t, w2_ref[...], preferred_element_type=jnp.float32) \
            + b2_ref[...]
        o = jnp.where(o >= 0, o, o * SLOPE)
        y_ref[0] = o[:, :H // 2]
        y_ref[1] = o[:, H // 2:]

    x_spec = (pl.BlockSpec((BN, 128), lambda i: (i, 0)) if first
              else pl.BlockSpec((2, BN, 128), lambda i: (0, i, 0)))
    return pl.pallas_call(
        body,
        grid=grid,
        in_specs=[
            x_spec,
            pl.BlockSpec((2, BN, 128), lambda i: (0, i, 0)),
            pl.BlockSpec((D, H), lambda i: (0, 0)),
            pl.BlockSpec((1, H), lambda i: (0, 0)),
            pl.BlockSpec((H, H), lambda i: (0, 0)),
            pl.BlockSpec((1, H), lambda i: (0, 0)),
        ],
        out_specs=pl.BlockSpec((2, BN, H // 2), lambda i: (0, i, 0)),
        out_shape=jax.ShapeDtypeStruct((2, N, H // 2), jnp.float32),
    )(xs, ag, W1, b1.reshape(1, H), W2, b2.reshape(1, H))


def _final_call(xs, batch2d, lin_W, lin_b, ln_g, ln_b, p1_W, p1_b, p2_W, p2_b):
    BN = 1000
    GRID = N // BN

    def body(x_ref, b_ref, linW_ref, linb_ref, lng_ref, lnb_ref,
             p1W_ref, p1b_ref, p2W_ref, p2b_ref, z_ref, pooled, cnt):
        i = pl.program_id(0)
        xb = jnp.concatenate([x_ref[0], x_ref[1]], axis=1)
        oh = (b_ref[...] == lax.broadcasted_iota(jnp.int32, (BN, G), 1)
              ).astype(jnp.float32)
        pooled_blk = lax.dot_general(
            oh, xb, (((0,), (0,)), ((), ())),
            preferred_element_type=jnp.float32)
        cnt_blk = lax.dot_general(
            oh, jnp.ones((BN, 8), jnp.float32), (((0,), (0,)), ((), ())),
            preferred_element_type=jnp.float32)

        @pl.when(i == 0)
        def _():
            pooled[...] = pooled_blk
            cnt[...] = cnt_blk

        @pl.when(i > 0)
        def _():
            pooled[...] += pooled_blk
            cnt[...] += cnt_blk

        @pl.when(i == GRID - 1)
        def _():
            p = jnp.dot(pooled[...], linW_ref[...],
                        preferred_element_type=jnp.float32) \
                + cnt[...][:, :1] * linb_ref[...]
            mu = jnp.mean(p, axis=1, keepdims=True)
            var = jnp.mean((p - mu) ** 2, axis=1, keepdims=True)
            hh = (p - mu) / jnp.sqrt(var + 1e-5) * lng_ref[...] + lnb_ref[...]
            a = jnp.dot(hh, p1W_ref[...],
                        preferred_element_type=jnp.float32) + p1b_ref[...]
            a = jnp.where(a >= 0, a, a * SLOPE)
            z_ref[...] = jnp.dot(a, p2W_ref[...],
                                 preferred_element_type=jnp.float32) \
                + p2b_ref[...]

    return pl.pallas_call(
        body,
        grid=(GRID,),
        in_specs=[
            pl.BlockSpec((2, BN, H // 2), lambda i: (0, i, 0)),
            pl.BlockSpec((BN, 1), lambda i: (i, 0)),
            pl.BlockSpec((H, OUT), lambda i: (0, 0)),
            pl.BlockSpec((1, OUT), lambda i: (0, 0)),
            pl.BlockSpec((1, OUT), lambda i: (0, 0)),
            pl.BlockSpec((1, OUT), lambda i: (0, 0)),
            pl.BlockSpec((OUT, 256), lambda i: (0, 0)),
            pl.BlockSpec((1, 256), lambda i: (0, 0)),
            pl.BlockSpec((256, OUT), lambda i: (0, 0)),
            pl.BlockSpec((1, OUT), lambda i: (0, 0)),
        ],
        out_specs=pl.BlockSpec((G, OUT), lambda i: (0, 0)),
        out_shape=jax.ShapeDtypeStruct((G, OUT), jnp.float32),
        scratch_shapes=[
            pltpu.VMEM((G, 256), jnp.float32),
            pltpu.VMEM((G, 8), jnp.float32),
        ],
    )(xs, batch2d, lin_W, lin_b.reshape(1, OUT), ln_g.reshape(1, OUT),
      ln_b.reshape(1, OUT), p1_W, p1_b.reshape(1, 256), p2_W,
      p2_b.reshape(1, OUT))


def kernel(x, edge_index, batch,
           gin1_W1, gin1_b1, gin1_W2, gin1_b2,
           gin2_W1, gin2_b1, gin2_W2, gin2_b2,
           gin3_W1, gin3_b1, gin3_W2, gin3_b2,
           gin4_W1, gin4_b1, gin4_W2, gin4_b2,
           gin5_W1, gin5_b1, gin5_W2, gin5_b2,
           lin_W, lin_b, ln_g, ln_b,
           p1_W, p1_b, p2_W, p2_b):
    src = edge_index[0]
    dst = edge_index[1]
    pad1 = E_PAD1 - E
    src1 = jnp.concatenate([src, jnp.zeros((pad1,), jnp.int32)])
    dst1 = jnp.concatenate([dst, jnp.full((pad1,), DUMMY, jnp.int32)])
    src1 = src1.reshape(2, TILES, STEPS1, EB)
    dst1 = dst1.reshape(2, TILES, STEPS1, EB)
    pad = E_PAD - E
    src_p = jnp.concatenate([src, jnp.zeros((pad,), jnp.int32)])
    dst_p = jnp.concatenate([dst, jnp.full((pad,), DUMMY, jnp.int32)])
    src2 = jnp.stack([src_p, src_p + N]).reshape(2, TILES, STEPS, EB)
    dst2 = dst_p.reshape(TILES, STEPS, EB)

    ag = _sc_agg_edge(x, src1, dst1)
    xs = _mlp_call(x, ag, gin1_W1, gin1_b1, gin1_W2, gin1_b2, first=True)

    for (W1, b1, W2, b2) in ((gin2_W1, gin2_b1, gin2_W2, gin2_b2),
                             (gin3_W1, gin3_b1, gin3_W2, gin3_b2),
                             (gin4_W1, gin4_b1, gin4_W2, gin4_b2),
                             (gin5_W1, gin5_b1, gin5_W2, gin5_b2)):
        ag = _sc_agg_feat(xs.reshape(2 * N, 128), src2, dst2)
        xs = _mlp_call(xs, ag, W1, b1, W2, b2)

    return _final_call(xs, batch.reshape(N, 1), lin_W, lin_b, ln_g, ln_b,
                       p1_W, p1_b, p2_W, p2_b)
